# Initial kernel scaffold; baseline (speedup 1.0000x reference)
import jax
import jax.numpy as jnp
from jax import lax
from jax.experimental import pallas as pl
from jax.experimental.pallas import tpu as pltpu
from jax.experimental.pallas import tpu_sc as plsc

B = 16384
CTX = 50
D = 64
HID = 128

NC = 2
NS = 16
NW = NC * NS

SAMPLES_PER_W = B // NW
PAIRS_PER_W = SAMPLES_PER_W // 2
IDX_PER_PAIR = 2 * CTX


def _pool_body(idx_hbm, table_hbm, out_hbm, idx_v, rows_v, acc_v, sem):
    wid = lax.axis_index("s") * NC + lax.axis_index("c")
    pair_base = wid * PAIRS_PER_W

    pltpu.sync_copy(idx_hbm.at[pl.ds(pair_base, PAIRS_PER_W)], idx_v)

    def chunk(c, carry):
        pltpu.async_copy(table_hbm.at[idx_v.at[c]], rows_v, sem).wait()
        for s in range(2):
            for seg in range(D // 16):
                v = rows_v[CTX * s, pl.ds(16 * seg, 16)]
                for r in range(1, CTX):
                    v = v + rows_v[CTX * s + r, pl.ds(16 * seg, 16)]
                acc_v[2 * c + s, pl.ds(16 * seg, 16)] = v
        return carry

    lax.fori_loop(0, PAIRS_PER_W, chunk, 0)
    pltpu.sync_copy(acc_v, out_hbm.at[pl.ds(wid * SAMPLES_PER_W, SAMPLES_PER_W)])


def _pooled_embeddings(idx2d, table):
    kern = pl.kernel(
        _pool_body,
        out_type=jax.ShapeDtypeStruct((B, D), jnp.float32),
        mesh=plsc.VectorSubcoreMesh(core_axis_name="c", subcore_axis_name="s"),
        scratch_types=[
            pltpu.VMEM((PAIRS_PER_W, IDX_PER_PAIR), jnp.int32),
            pltpu.VMEM((IDX_PER_PAIR, D), jnp.float32),
            pltpu.VMEM((SAMPLES_PER_W, D), jnp.float32),
            pltpu.SemaphoreType.DMA,
        ],
    )
    return kern(idx2d, table)


def _mlp_body(x_ref, w1_ref, b1_ref, w2_ref, b2_ref, o_ref):
    h = jnp.dot(x_ref[...], w1_ref[...], preferred_element_type=jnp.float32)
    h = jnp.maximum(h + b1_ref[...], 0.0)
    logits = jnp.dot(h, w2_ref[...], preferred_element_type=jnp.float32)
    logits = logits + b2_ref[...]
    m = jnp.max(logits, axis=1, keepdims=True)
    lse = jnp.log(jnp.sum(jnp.exp(logits - m), axis=1, keepdims=True)) + m
    o_ref[...] = logits - lse


def _mlp(embeds, W1, b1, W2, b2):
    bs = 2048
    return pl.pallas_call(
        _mlp_body,
        grid=(B // bs,),
        in_specs=[
            pl.BlockSpec((bs, D), lambda i: (i, 0)),
            pl.BlockSpec((D, HID), lambda i: (0, 0)),
            pl.BlockSpec((1, HID), lambda i: (0, 0)),
            pl.BlockSpec((HID, 2), lambda i: (0, 0)),
            pl.BlockSpec((1, 2), lambda i: (0, 0)),
        ],
        out_specs=pl.BlockSpec((bs, 2), lambda i: (i, 0)),
        out_shape=jax.ShapeDtypeStruct((B, 2), jnp.float32),
    )(embeds, W1, b1.reshape(1, HID), W2, b2.reshape(1, 2))


@jax.jit
def kernel(inputs, table, W1, b1, W2, b2):
    idx2d = inputs.reshape(B // 2, IDX_PER_PAIR).astype(jnp.int32)
    embeds = _pooled_embeddings(idx2d, table)
    return _mlp(embeds, W1, b1, W2, b2)


# speedup vs baseline: 2.0390x; 2.0390x over previous
import jax
import jax.numpy as jnp
from jax import lax
from jax.experimental import pallas as pl
from jax.experimental.pallas import tpu as pltpu
from jax.experimental.pallas import tpu_sc as plsc

B = 16384
CTX = 50
D = 64
HID = 128

NC = 2
NS = 16
NW = NC * NS

SAMPLES_PER_W = B // NW
PAIRS_PER_W = SAMPLES_PER_W // 2
IDX_PER_PAIR = 2 * CTX


def _pool_body(idx_hbm, table_hbm, out_hbm, idx_v, rows_v, acc_v, sem):
    wid = lax.axis_index("s") * NC + lax.axis_index("c")
    pair_base = wid * PAIRS_PER_W

    pltpu.sync_copy(idx_hbm.at[pl.ds(pair_base, PAIRS_PER_W)], idx_v)

    def chunk(c, carry):
        pltpu.async_copy(table_hbm.at[idx_v.at[c]], rows_v, sem).wait()
        for s in range(2):
            for seg in range(D // 16):
                v = rows_v[CTX * s, pl.ds(16 * seg, 16)]
                for r in range(1, CTX):
                    v = v + rows_v[CTX * s + r, pl.ds(16 * seg, 16)]
                acc_v[2 * c + s, pl.ds(16 * seg, 16)] = v
        return carry

    lax.fori_loop(0, PAIRS_PER_W, chunk, 0)
    pltpu.sync_copy(acc_v, out_hbm.at[pl.ds(wid * SAMPLES_PER_W, SAMPLES_PER_W)])


def _pooled_embeddings(idx2d, table):
    kern = pl.kernel(
        _pool_body,
        out_type=jax.ShapeDtypeStruct((B, D), jnp.float32),
        mesh=plsc.VectorSubcoreMesh(
            core_axis_name="c", subcore_axis_name="s",
            num_cores=NC, num_subcores=NS,
        ),
        scratch_types=[
            pltpu.VMEM((PAIRS_PER_W, IDX_PER_PAIR), jnp.int32),
            pltpu.VMEM((IDX_PER_PAIR, D), jnp.float32),
            pltpu.VMEM((SAMPLES_PER_W, D), jnp.float32),
            pltpu.SemaphoreType.DMA,
        ],
        compiler_params=pltpu.CompilerParams(use_tc_tiling_on_sc=False),
    )
    return kern(idx2d, table)


def _mlp_body(x_ref, w1_ref, b1_ref, w2_ref, b2_ref, o_ref):
    h = jnp.dot(x_ref[...], w1_ref[...], preferred_element_type=jnp.float32)
    h = jnp.maximum(h + b1_ref[...], 0.0)
    logits = jnp.dot(h, w2_ref[...], preferred_element_type=jnp.float32)
    logits = logits + b2_ref[...]
    m = jnp.max(logits, axis=1, keepdims=True)
    lse = jnp.log(jnp.sum(jnp.exp(logits - m), axis=1, keepdims=True)) + m
    o_ref[...] = logits - lse


def _mlp(embeds, W1, b1, W2, b2):
    bs = 2048
    return pl.pallas_call(
        _mlp_body,
        grid=(B // bs,),
        in_specs=[
            pl.BlockSpec((bs, D), lambda i: (i, 0)),
            pl.BlockSpec((D, HID), lambda i: (0, 0)),
            pl.BlockSpec((1, HID), lambda i: (0, 0)),
            pl.BlockSpec((HID, 2), lambda i: (0, 0)),
            pl.BlockSpec((1, 2), lambda i: (0, 0)),
        ],
        out_specs=pl.BlockSpec((bs, 2), lambda i: (i, 0)),
        out_shape=jax.ShapeDtypeStruct((B, 2), jnp.float32),
    )(embeds, W1, b1.reshape(1, HID), W2, b2.reshape(1, 2))


@jax.jit
def kernel(inputs, table, W1, b1, W2, b2):
    idx2d = inputs.reshape(B // 2, IDX_PER_PAIR).astype(jnp.int32)
    embeds = _pooled_embeddings(idx2d, table)
    return _mlp(embeds, W1, b1, W2, b2)


# speedup vs baseline: 2.0830x; 1.0216x over previous
import jax
import jax.numpy as jnp
from jax import lax
from jax.experimental import pallas as pl
from jax.experimental.pallas import tpu as pltpu
from jax.experimental.pallas import tpu_sc as plsc

B = 16384
CTX = 50
D = 64
HID = 128

NC = 2
NS = 16
NW = NC * NS

SAMPLES_PER_W = B // NW
PAIRS_PER_W = SAMPLES_PER_W // 2
IDX_PER_PAIR = 2 * CTX


NBUF = 4


def _pool_body(idx_hbm, table_hbm, out_hbm, idx_v, acc_v, *rest):
    rows_bufs, sems = rest[:NBUF], rest[NBUF:]
    wid = lax.axis_index("s") * NC + lax.axis_index("c")
    pair_base = wid * PAIRS_PER_W

    pltpu.sync_copy(idx_hbm.at[pl.ds(pair_base, PAIRS_PER_W)], idx_v)

    for b in range(NBUF):
        pltpu.async_copy(table_hbm.at[idx_v.at[b]], rows_bufs[b], sems[b])

    def group(i, carry):
        for b in range(NBUF):
            g = NBUF * i + b
            rows_v, sem = rows_bufs[b], sems[b]
            pltpu.make_async_copy(
                table_hbm.at[idx_v.at[g]], rows_v, sem).wait()
            for s in range(2):
                for seg in range(D // 16):
                    v = rows_v[CTX * s, pl.ds(16 * seg, 16)]
                    for r in range(1, CTX):
                        v = v + rows_v[CTX * s + r, pl.ds(16 * seg, 16)]
                    acc_v[2 * g + s, pl.ds(16 * seg, 16)] = v

            @pl.when(g + NBUF < PAIRS_PER_W)
            def _():
                pltpu.async_copy(
                    table_hbm.at[idx_v.at[g + NBUF]], rows_v, sem)
        return carry

    lax.fori_loop(0, PAIRS_PER_W // NBUF, group, 0)
    pltpu.sync_copy(acc_v, out_hbm.at[pl.ds(wid * SAMPLES_PER_W, SAMPLES_PER_W)])


def _pooled_embeddings(idx2d, table):
    kern = pl.kernel(
        _pool_body,
        out_type=jax.ShapeDtypeStruct((B, D), jnp.float32),
        mesh=plsc.VectorSubcoreMesh(
            core_axis_name="c", subcore_axis_name="s",
            num_cores=NC, num_subcores=NS,
        ),
        scratch_types=(
            [
                pltpu.VMEM((PAIRS_PER_W, IDX_PER_PAIR), jnp.int32),
                pltpu.VMEM((SAMPLES_PER_W, D), jnp.float32),
            ]
            + [pltpu.VMEM((IDX_PER_PAIR, D), jnp.float32)] * NBUF
            + [pltpu.SemaphoreType.DMA] * NBUF
        ),
        compiler_params=pltpu.CompilerParams(use_tc_tiling_on_sc=False),
    )
    return kern(idx2d, table)


def _mlp_body(x_ref, w1_ref, b1_ref, w2_ref, b2_ref, o_ref):
    h = jnp.dot(x_ref[...], w1_ref[...], preferred_element_type=jnp.float32)
    h = jnp.maximum(h + b1_ref[...], 0.0)
    logits = jnp.dot(h, w2_ref[...], preferred_element_type=jnp.float32)
    logits = logits + b2_ref[...]
    m = jnp.max(logits, axis=1, keepdims=True)
    lse = jnp.log(jnp.sum(jnp.exp(logits - m), axis=1, keepdims=True)) + m
    o_ref[...] = logits - lse


def _mlp(embeds, W1, b1, W2, b2):
    bs = 2048
    return pl.pallas_call(
        _mlp_body,
        grid=(B // bs,),
        in_specs=[
            pl.BlockSpec((bs, D), lambda i: (i, 0)),
            pl.BlockSpec((D, HID), lambda i: (0, 0)),
            pl.BlockSpec((1, HID), lambda i: (0, 0)),
            pl.BlockSpec((HID, 2), lambda i: (0, 0)),
            pl.BlockSpec((1, 2), lambda i: (0, 0)),
        ],
        out_specs=pl.BlockSpec((bs, 2), lambda i: (i, 0)),
        out_shape=jax.ShapeDtypeStruct((B, 2), jnp.float32),
    )(embeds, W1, b1.reshape(1, HID), W2, b2.reshape(1, 2))


@jax.jit
def kernel(inputs, table, W1, b1, W2, b2):
    idx2d = inputs.reshape(B // 2, IDX_PER_PAIR).astype(jnp.int32)
    embeds = _pooled_embeddings(idx2d, table)
    return _mlp(embeds, W1, b1, W2, b2)
